# baseline (speedup 1.0000x reference)
import jax
import jax.numpy as jnp
from jax import lax
from jax.experimental import pallas as pl
from jax.experimental.pallas import tpu as pltpu
from jax.experimental.pallas import tpu_sc as plsc

B = 4096
K = 32

_info = plsc.get_sparse_core_info()
_NC = _info.num_cores
_NS = _info.num_subcores
_NW = _NC * _NS
_BPW = B // _NW
_L = 16


def _gather_body(user_hbm, item_hbm, utab_hbm, itab_hbm, uout_hbm, iout_hbm,
                 uidx_v, iidx_v, urows_v, irows_v, sem):
  wid = lax.axis_index("s") * _NC + lax.axis_index("c")
  base = wid * _BPW
  pltpu.sync_copy(user_hbm.at[pl.ds(base, _BPW)], uidx_v)
  pltpu.sync_copy(item_hbm.at[pl.ds(base, _BPW)], iidx_v)
  lanes = lax.iota(jnp.int32, _L)

  for c in range(_BPW // _L):
    uvec = uidx_v[pl.ds(c * _L, _L)]
    ivec = iidx_v[pl.ds(c * _L, _L)]
    for l in range(_L):
      urow = jnp.sum(jnp.where(lanes == l, uvec, 0))
      irow = jnp.sum(jnp.where(lanes == l, ivec, 0))
      dst = c * _L + l
      pltpu.async_copy(
          utab_hbm.at[pl.ds(urow, 1), :], urows_v.at[pl.ds(dst, 1), :], sem)
      pltpu.async_copy(
          itab_hbm.at[pl.ds(irow, 1), :], irows_v.at[pl.ds(dst, 1), :], sem)

  pltpu.make_async_copy(utab_hbm.at[pl.ds(0, _BPW), :], urows_v, sem).wait()
  pltpu.make_async_copy(itab_hbm.at[pl.ds(0, _BPW), :], irows_v, sem).wait()
  pltpu.sync_copy(urows_v, uout_hbm.at[pl.ds(base, _BPW), :])
  pltpu.sync_copy(irows_v, iout_hbm.at[pl.ds(base, _BPW), :])


_gather = pl.kernel(
    _gather_body,
    out_type=(
        jax.ShapeDtypeStruct((B, K), jnp.float32),
        jax.ShapeDtypeStruct((B, K), jnp.float32),
    ),
    mesh=plsc.VectorSubcoreMesh(core_axis_name="c", subcore_axis_name="s"),
    scratch_types=[
        pltpu.VMEM((_BPW,), jnp.int32),
        pltpu.VMEM((_BPW,), jnp.int32),
        pltpu.VMEM((_BPW, K), jnp.float32),
        pltpu.VMEM((_BPW, K), jnp.float32),
        pltpu.SemaphoreType.DMA,
    ],
    compiler_params=pltpu.CompilerParams(needs_layout_passes=False),
)


_BM = 256
_NT = B // _BM


def _residual_body(rating_ref, u_ref, v_ref, out_ref):
  preds = lax.dot_general(
      u_ref[...], v_ref[...],
      dimension_numbers=(((1,), (1,)), ((), ())),
      preferred_element_type=jnp.float32)
  out_ref[...] = rating_ref[...] - preds


def _residual(rating, u_emb, i_emb):
  return pl.pallas_call(
      _residual_body,
      grid=(_NT,),
      in_specs=[
          pl.BlockSpec((_BM, B), lambda i: (i, 0)),
          pl.BlockSpec((_BM, K), lambda i: (i, 0)),
          pl.BlockSpec((B, K), lambda i: (0, 0)),
      ],
      out_specs=pl.BlockSpec((_BM, B), lambda i: (i, 0)),
      out_shape=jax.ShapeDtypeStruct((B, B), jnp.float32),
  )(rating, u_emb, i_emb)


@jax.jit
def kernel(user, item, rating, user_table, item_table):
  u_emb, i_emb = _gather(user.astype(jnp.int32), item.astype(jnp.int32),
                         user_table, item_table)
  return _residual(rating, u_emb, i_emb)


# speedup vs baseline: 3.1593x; 3.1593x over previous
import jax
import jax.numpy as jnp
from jax import lax
from jax.experimental import pallas as pl
from jax.experimental.pallas import tpu as pltpu
from jax.experimental.pallas import tpu_sc as plsc

B = 4096
K = 32

_info = plsc.get_sparse_core_info()
_NC = _info.num_cores
_NS = _info.num_subcores
_NW = _NC * _NS
_BPW = B // _NW


def _item_gather_body(item_hbm, itab_hbm, iout_hbm, iidx_v, irows_v, sem):
  wid = lax.axis_index("s") * _NC + lax.axis_index("c")
  base = wid * _BPW
  pltpu.sync_copy(item_hbm.at[pl.ds(base, _BPW)], iidx_v)
  pltpu.async_copy(itab_hbm.at[iidx_v], irows_v, sem).wait()
  pltpu.sync_copy(irows_v, iout_hbm.at[pl.ds(base, _BPW)])


_item_gather = pl.kernel(
    _item_gather_body,
    out_type=jax.ShapeDtypeStruct((B, K), jnp.float32),
    mesh=plsc.VectorSubcoreMesh(core_axis_name="c", subcore_axis_name="s"),
    scratch_types=[
        pltpu.VMEM((_BPW,), jnp.int32),
        pltpu.VMEM((_BPW, K), jnp.float32),
        pltpu.SemaphoreType.DMA,
    ],
    compiler_params=pltpu.CompilerParams(use_tc_tiling_on_sc=False),
)


_BM = 512
_NT = B // _BM


def _residual_body(rating_ref, u_ref, v_ref, out_ref):
  preds = lax.dot_general(
      u_ref[...], v_ref[...],
      dimension_numbers=(((1,), (1,)), ((), ())),
      preferred_element_type=jnp.float32)
  out_ref[...] = rating_ref[...] - preds


def _residual(rating, u_emb, i_emb):
  return pl.pallas_call(
      _residual_body,
      grid=(_NT,),
      in_specs=[
          pl.BlockSpec((_BM, B), lambda i: (i, 0)),
          pl.BlockSpec((_BM, K), lambda i: (i, 0)),
          pl.BlockSpec((B, K), lambda i: (0, 0)),
      ],
      out_specs=pl.BlockSpec((_BM, B), lambda i: (i, 0)),
      out_shape=jax.ShapeDtypeStruct((B, B), jnp.float32),
  )(rating, u_emb, i_emb)


@jax.jit
def kernel(user, item, rating, user_table, item_table):
  i_emb = _item_gather(item.astype(jnp.int32), item_table)
  u_emb = jnp.take(user_table, user, axis=0)
  return _residual(rating, u_emb, i_emb)


# speedup vs baseline: 3.2259x; 1.0211x over previous
import jax
import jax.numpy as jnp
from jax import lax
from jax.experimental import pallas as pl
from jax.experimental.pallas import tpu as pltpu
from jax.experimental.pallas import tpu_sc as plsc

B = 4096
K = 32

_info = plsc.get_sparse_core_info()
_NC = _info.num_cores
_NS = _info.num_subcores
_NW = _NC * _NS
_BPW = B // _NW


def _item_gather_body(item_hbm, itab_hbm, iout_hbm, iidx_v, irows_v, sem):
  wid = lax.axis_index("s") * _NC + lax.axis_index("c")
  base = wid * _BPW
  pltpu.sync_copy(item_hbm.at[pl.ds(base, _BPW)], iidx_v)
  pltpu.async_copy(itab_hbm.at[iidx_v], irows_v, sem).wait()
  pltpu.sync_copy(irows_v, iout_hbm.at[pl.ds(base, _BPW)])


_item_gather = pl.kernel(
    _item_gather_body,
    out_type=jax.ShapeDtypeStruct((B, K), jnp.float32),
    mesh=plsc.VectorSubcoreMesh(core_axis_name="c", subcore_axis_name="s"),
    scratch_types=[
        pltpu.VMEM((_BPW,), jnp.int32),
        pltpu.VMEM((_BPW, K), jnp.float32),
        pltpu.SemaphoreType.DMA,
    ],
    compiler_params=pltpu.CompilerParams(use_tc_tiling_on_sc=False),
)


_BM = 512
_NT = B // _BM


def _residual_body(rating_ref, u_ref, v_ref, out_ref):
  preds = lax.dot_general(
      u_ref[...], v_ref[...],
      dimension_numbers=(((1,), (1,)), ((), ())),
      preferred_element_type=jnp.float32)
  out_ref[...] = rating_ref[...] - preds


def _residual(rating, u_emb, i_emb):
  return pl.pallas_call(
      _residual_body,
      grid=(_NT,),
      in_specs=[
          pl.BlockSpec((_BM, B), lambda i: (i, 0)),
          pl.BlockSpec((_BM, K), lambda i: (i, 0)),
          pl.BlockSpec((B, K), lambda i: (0, 0)),
      ],
      out_specs=pl.BlockSpec((_BM, B), lambda i: (i, 0)),
      out_shape=jax.ShapeDtypeStruct((B, B), jnp.float32),
  )(rating, u_emb, i_emb)


@jax.jit
def kernel(user, item, rating, user_table, item_table):
  i_emb = _item_gather(item.astype(jnp.int32), item_table)
  u_emb = user_table.at[user].get(mode="promise_in_bounds")
  return _residual(rating, u_emb, i_emb)


# speedup vs baseline: 3.2285x; 1.0008x over previous
import jax
import jax.numpy as jnp
from jax import lax
from jax.experimental import pallas as pl
from jax.experimental.pallas import tpu as pltpu
from jax.experimental.pallas import tpu_sc as plsc

B = 4096
K = 32

_info = plsc.get_sparse_core_info()
_NC = _info.num_cores
_NS = _info.num_subcores
_NW = _NC * _NS
_BPW = B // _NW


def _item_gather_body(item_hbm, itab_hbm, iout_hbm, iidx_v, irows_v, sem):
  wid = lax.axis_index("s") * _NC + lax.axis_index("c")
  base = wid * _BPW
  pltpu.sync_copy(item_hbm.at[pl.ds(base, _BPW)], iidx_v)
  pltpu.async_copy(itab_hbm.at[iidx_v], irows_v, sem).wait()
  pltpu.sync_copy(irows_v, iout_hbm.at[pl.ds(base, _BPW)])


_item_gather = pl.kernel(
    _item_gather_body,
    out_type=jax.ShapeDtypeStruct((B, K), jnp.float32),
    mesh=plsc.VectorSubcoreMesh(core_axis_name="c", subcore_axis_name="s"),
    scratch_types=[
        pltpu.VMEM((_BPW,), jnp.int32),
        pltpu.VMEM((_BPW, K), jnp.float32),
        pltpu.SemaphoreType.DMA,
    ],
    compiler_params=pltpu.CompilerParams(use_tc_tiling_on_sc=False),
)


_BM = 512
_NT = B // _BM


def _residual_body(rating_ref, u_ref, v_ref, out_ref):
  preds = lax.dot_general(
      u_ref[...], v_ref[...],
      dimension_numbers=(((1,), (1,)), ((), ())),
      preferred_element_type=jnp.float32)
  out_ref[...] = rating_ref[...] - preds


def _residual(rating, u_emb, i_emb):
  return pl.pallas_call(
      _residual_body,
      grid=(_NT,),
      in_specs=[
          pl.BlockSpec((_BM, B), lambda i: (i, 0)),
          pl.BlockSpec((_BM, K), lambda i: (i, 0)),
          pl.BlockSpec((B, K), lambda i: (0, 0)),
      ],
      out_specs=pl.BlockSpec((_BM, B), lambda i: (i, 0)),
      out_shape=jax.ShapeDtypeStruct((B, B), jnp.float32),
  )(rating, u_emb, i_emb)


@jax.jit
def kernel(user, item, rating, user_table, item_table):
  i_emb = _item_gather(item.astype(jnp.int32), item_table)
  u_emb = lax.gather(
      user_table[None], user.astype(jnp.int32)[None, :, None],
      dimension_numbers=lax.GatherDimensionNumbers(
          offset_dims=(2,), collapsed_slice_dims=(1,),
          start_index_map=(1,), operand_batching_dims=(0,),
          start_indices_batching_dims=(0,)),
      slice_sizes=(1, 1, K), mode="promise_in_bounds")[0]
  return _residual(rating, u_emb, i_emb)
